# baseline (speedup 1.0000x reference)
import functools

import jax
import jax.numpy as jnp
from jax import lax
from jax.experimental import pallas as pl
from jax.experimental.pallas import tpu as pltpu
from jax.experimental.pallas import tpu_sc as plsc

N_NODES = 10000
N_EDGES = 320000
D = 128
NC = 2
NS = 16
NW = NC * NS
CH = 80
EPS = N_EDGES // NW
TPS = EPS // CH
HALF = (TPS + 1) // 2
RPT = 624
RTAIL_BASE = NS * RPT
RTAIL = N_NODES - RTAIL_BASE

_MESH = plsc.VectorSubcoreMesh(core_axis_name="c", subcore_axis_name="s")


def _zero_rows(ref, nrows, width):
    def body(i, carry):
        for k in range(width // 16):
            ref[i, pl.ds(k * 16, 16)] = jnp.zeros((16,), jnp.float32)
        return carry
    lax.fori_loop(0, nrows, body, 0)


def _fill_ones_w(ref, nrows, width):
    def body(i, carry):
        for k in range(width // 16):
            ref[i, pl.ds(k * 16, 16)] = jnp.full((16,), 1.0, jnp.float32)
        return carry
    lax.fori_loop(0, nrows, body, 0)


def _copy_zero_region(zbuf, dst, base, total, bufrows):
    nfull = total // bufrows
    rem = total % bufrows
    for r in range(nfull):
        pltpu.sync_copy(zbuf, dst.at[pl.ds(base + r * bufrows, bufrows)])
    if rem:
        pltpu.sync_copy(zbuf.at[pl.ds(0, rem)],
                        dst.at[pl.ds(base + nfull * bufrows, rem)])


EPS2 = N_EDGES // NS
WIN = EPS2 // 2
TPSW = WIN // CH
HALFW = (TPSW + 1) // 2


def _segdeg_body(tbl, srci, dsti, gd_out,
                 ixs, ixd, rb0, rb1, acc, sg0, sg1):
    c = lax.axis_index("c")
    s = lax.axis_index("s")
    ebase = s * EPS2
    base = s * RPT

    _zero_rows(rb0, CH, D)
    _copy_zero_region(rb0, acc, base, RPT, CH)

    @pl.when(s == NS - 1)
    def _():
        pltpu.sync_copy(rb0.at[pl.ds(0, RTAIL)],
                        acc.at[pl.ds(RTAIL_BASE, RTAIL)])

    @pl.when(c == 1)
    def _():
        _fill_ones_w(rb1, CH, D)

    plsc.subcore_barrier()

    rbufs = (rb0, rb1)
    sems = (sg0, sg1)

    def issue(chunk, slot):
        pltpu.async_copy(tbl.at[ixs.at[pl.ds(chunk * CH, CH)]],
                         rbufs[slot], sems[slot])

    def drain(slot):
        pltpu.make_async_copy(tbl.at[pl.ds(0, CH)], rbufs[slot],
                              sems[slot]).wait()

    def scat(chunk, slot):
        pltpu.sync_copy(rbufs[slot],
                        acc.at[ixd.at[pl.ds(chunk * CH, CH)]], add=True)

    @pl.when(c == 0)
    def _():
        for win in range(2):
            pltpu.sync_copy(srci.at[pl.ds(ebase + win * WIN, WIN)], ixs)
            pltpu.sync_copy(dsti.at[pl.ds(ebase + win * WIN, WIN)], ixd)
            issue(0, 0)

            def body(i, carry):
                c0 = 2 * i

                @pl.when(c0 + 1 < TPSW)
                def _():
                    issue(c0 + 1, 1)

                drain(0)
                scat(c0, 0)

                @pl.when(c0 + 1 < TPSW)
                def _():
                    @pl.when(c0 + 2 < TPSW)
                    def _():
                        issue(c0 + 2, 0)

                    drain(1)
                    scat(c0 + 1, 1)

                return carry

            lax.fori_loop(0, HALFW, body, 0)

    @pl.when(c == 1)
    def _():
        for win in range(2):
            pltpu.sync_copy(dsti.at[pl.ds(ebase + win * WIN, WIN)], ixd)

            def step(t, carry):
                pltpu.sync_copy(rb1, acc.at[ixd.at[pl.ds(t * CH, CH)]],
                                add=True)
                return carry

            lax.fori_loop(0, TPSW, step, 0)

    plsc.subcore_barrier()
    pltpu.sync_copy(acc.at[pl.ds(base, RPT)], gd_out.at[c, pl.ds(base, RPT)])

    @pl.when(s == NS - 1)
    def _():
        pltpu.sync_copy(acc.at[pl.ds(RTAIL_BASE, RTAIL)],
                        gd_out.at[c, pl.ds(RTAIL_BASE, RTAIL)])


def _seg_body(tbl, srci, dsti, g_out,
              ixs, ixd, rb0, rb1, acc, sg0, sg1):
    c = lax.axis_index("c")
    s = lax.axis_index("s")
    w = s * NC + c
    ebase = w * EPS
    base = s * RPT
    pltpu.sync_copy(srci.at[pl.ds(ebase, EPS)], ixs)
    pltpu.sync_copy(dsti.at[pl.ds(ebase, EPS)], ixd)
    _zero_rows(rb0, CH, D)
    _copy_zero_region(rb0, acc, base, RPT, CH)

    @pl.when(s == NS - 1)
    def _():
        pltpu.sync_copy(rb0.at[pl.ds(0, RTAIL)],
                        acc.at[pl.ds(RTAIL_BASE, RTAIL)])

    plsc.subcore_barrier()

    rbufs = (rb0, rb1)
    sems = (sg0, sg1)

    def issue(chunk, slot):
        pltpu.async_copy(tbl.at[ixs.at[pl.ds(chunk * CH, CH)]],
                         rbufs[slot], sems[slot])

    def drain(slot):
        pltpu.make_async_copy(tbl.at[pl.ds(0, CH)], rbufs[slot],
                              sems[slot]).wait()

    def scat(chunk, slot):
        pltpu.sync_copy(rbufs[slot],
                        acc.at[ixd.at[pl.ds(chunk * CH, CH)]], add=True)

    issue(0, 0)

    def body(i, carry):
        c0 = 2 * i

        @pl.when(c0 + 1 < TPS)
        def _():
            issue(c0 + 1, 1)

        drain(0)
        scat(c0, 0)

        @pl.when(c0 + 1 < TPS)
        def _():
            @pl.when(c0 + 2 < TPS)
            def _():
                issue(c0 + 2, 0)

            drain(1)
            scat(c0 + 1, 1)

        return carry

    lax.fori_loop(0, HALF, body, 0)
    plsc.subcore_barrier()
    pltpu.sync_copy(acc.at[pl.ds(base, RPT)], g_out.at[c, pl.ds(base, RPT)])

    @pl.when(s == NS - 1)
    def _():
        pltpu.sync_copy(acc.at[pl.ds(RTAIL_BASE, RTAIL)],
                        g_out.at[c, pl.ds(RTAIL_BASE, RTAIL)])


_segdeg_kernel = pl.kernel(
    _segdeg_body,
    out_type=[jax.ShapeDtypeStruct((NC, N_NODES, D), jnp.float32)],
    mesh=_MESH,
    scratch_types=[
        pltpu.VMEM((WIN,), jnp.int32),
        pltpu.VMEM((WIN,), jnp.int32),
        pltpu.VMEM((CH, D), jnp.float32),
        pltpu.VMEM((CH, D), jnp.float32),
        pltpu.VMEM_SHARED((N_NODES, D), jnp.float32),
        pltpu.SemaphoreType.DMA,
        pltpu.SemaphoreType.DMA,
    ],
    name="seg_deg",
)

_seg_sum = pl.kernel(
    _seg_body,
    out_type=[jax.ShapeDtypeStruct((NC, N_NODES, D), jnp.float32)],
    mesh=_MESH,
    scratch_types=[
        pltpu.VMEM((EPS,), jnp.int32),
        pltpu.VMEM((EPS,), jnp.int32),
        pltpu.VMEM((CH, D), jnp.float32),
        pltpu.VMEM((CH, D), jnp.float32),
        pltpu.VMEM_SHARED((N_NODES, D), jnp.float32),
        pltpu.SemaphoreType.DMA,
        pltpu.SemaphoreType.DMA,
    ],
    name="seg_sum",
)


def _edge_body(ts, td, srci, dsti, af_out, aq_out, bf_out, bq_out,
               ixs, ixd, ab0, ab1, bb0, bb1, qa, qb, sg0, sg1):
    c = lax.axis_index("c")
    s = lax.axis_index("s")
    w = s * NC + c
    ebase = w * EPS
    pltpu.sync_copy(srci.at[pl.ds(ebase, EPS)], ixs)
    pltpu.sync_copy(dsti.at[pl.ds(ebase, EPS)], ixd)

    abufs = (ab0, ab1)
    bbufs = (bb0, bb1)
    sems = (sg0, sg1)

    def issue(chunk, slot):
        isl = pl.ds(chunk * CH, CH)
        pltpu.async_copy(ts.at[ixs.at[isl]], abufs[slot], sems[slot])
        pltpu.async_copy(td.at[ixd.at[isl]], bbufs[slot], sems[slot])

    def drain(slot):
        pltpu.make_async_copy(ts.at[pl.ds(0, CH)], abufs[slot],
                              sems[slot]).wait()
        pltpu.make_async_copy(td.at[pl.ds(0, CH)], bbufs[slot],
                              sems[slot]).wait()

    rall = pl.ds(0, CH)

    def write(chunk, slot):
        osl = pl.ds(ebase + chunk * CH, CH)

        def pack(i, carry):
            qa[i, :] = abufs[slot][i, pl.ds(D, 16)]
            qb[i, :] = bbufs[slot][i, pl.ds(D, 16)]
            return carry

        lax.fori_loop(0, CH, pack, 0)
        pltpu.sync_copy(abufs[slot].at[rall, pl.ds(0, D)], af_out.at[osl])
        pltpu.sync_copy(qa, aq_out.at[osl])
        pltpu.sync_copy(bbufs[slot].at[rall, pl.ds(0, D)], bf_out.at[osl])
        pltpu.sync_copy(qb, bq_out.at[osl])

    issue(0, 0)

    def body(i, carry):
        c0 = 2 * i

        @pl.when(c0 + 1 < TPS)
        def _():
            issue(c0 + 1, 1)

        drain(0)
        write(c0, 0)

        @pl.when(c0 + 1 < TPS)
        def _():
            @pl.when(c0 + 2 < TPS)
            def _():
                issue(c0 + 2, 0)

            drain(1)
            write(c0 + 1, 1)

        return carry

    lax.fori_loop(0, HALF, body, 0)


_edge_gather = pl.kernel(
    _edge_body,
    out_type=[
        jax.ShapeDtypeStruct((N_EDGES, D), jnp.float32),
        jax.ShapeDtypeStruct((N_EDGES, 16), jnp.float32),
        jax.ShapeDtypeStruct((N_EDGES, D), jnp.float32),
        jax.ShapeDtypeStruct((N_EDGES, 16), jnp.float32),
    ],
    mesh=_MESH,
    scratch_types=[
        pltpu.VMEM((EPS,), jnp.int32),
        pltpu.VMEM((EPS,), jnp.int32),
        pltpu.VMEM((CH, 2 * D), jnp.float32),
        pltpu.VMEM((CH, 2 * D), jnp.float32),
        pltpu.VMEM((CH, 2 * D), jnp.float32),
        pltpu.VMEM((CH, 2 * D), jnp.float32),
        pltpu.VMEM((CH, 16), jnp.float32),
        pltpu.VMEM((CH, 16), jnp.float32),
        pltpu.SemaphoreType.DMA,
        pltpu.SemaphoreType.DMA,
    ],
    name="edge_gather",
)


_RN = 400
_NGRID = N_NODES // _RN
_EB = 512
_EGRID = N_EDGES // _EB


def _tc1_body(x_ref, gd_ref, ws_ref, wn_ref, b_ref, o_ref):
    deg = jnp.maximum(gd_ref[1, :, :1], 1.0)
    neigh = gd_ref[0] / deg
    h = (jnp.dot(x_ref[...], ws_ref[...], preferred_element_type=jnp.float32)
         + jnp.dot(neigh, wn_ref[...], preferred_element_type=jnp.float32)
         + b_ref[...])
    o_ref[...] = jnp.maximum(h, 0.0)


def _tc1(x, gd, ws, wn, b):
    return pl.pallas_call(
        _tc1_body,
        grid=(_NGRID,),
        in_specs=[
            pl.BlockSpec((_RN, D), lambda i: (i, 0)),
            pl.BlockSpec((NC, _RN, D), lambda i: (0, i, 0)),
            pl.BlockSpec((D, D), lambda i: (0, 0)),
            pl.BlockSpec((D, D), lambda i: (0, 0)),
            pl.BlockSpec((1, D), lambda i: (0, 0)),
        ],
        out_specs=pl.BlockSpec((_RN, D), lambda i: (i, 0)),
        out_shape=jax.ShapeDtypeStruct((N_NODES, D), jnp.float32),
    )(x, gd, ws, wn, b)


def _tc2_body(x_ref, h1_ref, g_ref, d_ref, ws_ref, wn_ref, b_ref,
              wni_ref, wnj_ref, wq1_ref, wq2_ref,
              ts_ref, td_ref):
    g = g_ref[0] + g_ref[1]
    deg = jnp.maximum(d_ref[1, :, :1], 1.0)
    neigh = g / deg
    h = (jnp.dot(h1_ref[...], ws_ref[...], preferred_element_type=jnp.float32)
         + jnp.dot(neigh, wn_ref[...], preferred_element_type=jnp.float32)
         + b_ref[...])
    x = x_ref[...]
    fni = jnp.dot(x, wni_ref[...], preferred_element_type=jnp.float32)
    fnj = jnp.dot(x, wnj_ref[...], preferred_element_type=jnp.float32)
    p1 = jnp.dot(h, wq1_ref[...], preferred_element_type=jnp.float32)
    p2 = jnp.dot(h, wq2_ref[...], preferred_element_type=jnp.float32)
    ts_ref[...] = jnp.concatenate([fni, p1], axis=1)
    td_ref[...] = jnp.concatenate([fnj, p2], axis=1)


def _tc2(x, h1, gp2, dp, ws2, wn2, b2, wni, wnj, wq1, wq2):
    return pl.pallas_call(
        _tc2_body,
        grid=(_NGRID,),
        in_specs=[
            pl.BlockSpec((_RN, D), lambda i: (i, 0)),
            pl.BlockSpec((_RN, D), lambda i: (i, 0)),
            pl.BlockSpec((NC, _RN, D), lambda i: (0, i, 0)),
            pl.BlockSpec((NC, _RN, D), lambda i: (0, i, 0)),
            pl.BlockSpec((D, D), lambda i: (0, 0)),
            pl.BlockSpec((D, D), lambda i: (0, 0)),
            pl.BlockSpec((1, D), lambda i: (0, 0)),
            pl.BlockSpec((D, D), lambda i: (0, 0)),
            pl.BlockSpec((D, D), lambda i: (0, 0)),
            pl.BlockSpec((D, D), lambda i: (0, 0)),
            pl.BlockSpec((D, D), lambda i: (0, 0)),
        ],
        out_specs=[
            pl.BlockSpec((_RN, 2 * D), lambda i: (i, 0)),
            pl.BlockSpec((_RN, 2 * D), lambda i: (i, 0)),
        ],
        out_shape=[
            jax.ShapeDtypeStruct((N_NODES, 2 * D), jnp.float32),
            jax.ShapeDtypeStruct((N_NODES, 2 * D), jnp.float32),
        ],
    )(x, h1, gp2, dp, ws2, wn2, b2, wni, wnj, wq1, wq2)


def _tc0_body(e_ref, wf_ref, bias_ref, z_ref):
    z_ref[...] = (jnp.dot(e_ref[...], wf_ref[...],
                          preferred_element_type=jnp.float32) + bias_ref[...])


def _tc0(e, wf, bias):
    return pl.pallas_call(
        _tc0_body,
        grid=(_EGRID,),
        in_specs=[
            pl.BlockSpec((_EB, D), lambda i: (i, 0)),
            pl.BlockSpec((D, D), lambda i: (0, 0)),
            pl.BlockSpec((1, D), lambda i: (0, 0)),
        ],
        out_specs=pl.BlockSpec((_EB, D), lambda i: (i, 0)),
        out_shape=jax.ShapeDtypeStruct((N_EDGES, D), jnp.float32),
    )(e, wf, bias)


def _tc3_body(z_ref, a_ref, b_ref, aq_ref, bq_ref, wp3_ref, bp_ref, o_ref):
    z = z_ref[...] + a_ref[...] + b_ref[...]
    f = jnp.maximum(z, 0.01 * z)
    o_ref[...] = (jnp.dot(f, wp3_ref[...], preferred_element_type=jnp.float32)
                  + aq_ref[...] + bq_ref[...] + bp_ref[...])


def _tc3(z1, a_e, b_e, aq, bq, wp3, bp):
    return pl.pallas_call(
        _tc3_body,
        grid=(_EGRID,),
        in_specs=[
            pl.BlockSpec((_EB, D), lambda i: (i, 0)),
            pl.BlockSpec((_EB, D), lambda i: (i, 0)),
            pl.BlockSpec((_EB, D), lambda i: (i, 0)),
            pl.BlockSpec((_EB, 16), lambda i: (i, 0)),
            pl.BlockSpec((_EB, 16), lambda i: (i, 0)),
            pl.BlockSpec((D, 16), lambda i: (0, 0)),
            pl.BlockSpec((1, 16), lambda i: (0, 0)),
        ],
        out_specs=pl.BlockSpec((_EB, 16), lambda i: (i, 0)),
        out_shape=jax.ShapeDtypeStruct((N_EDGES, 16), jnp.float32),
    )(z1, a_e, b_e, aq, bq, wp3, bp)


def kernel(x, e, edge_index, W_self1, W_neigh1, b1, W_self2, W_neigh2, b2,
           W_node_src, W_ni, W_fij, W_nj, egat_bias, attn, W_pred, b_pred):
    src = edge_index[0]
    dst = edge_index[1]
    z1 = _tc0(e, W_fij, egat_bias.reshape(1, D))
    (gd,) = _segdeg_kernel(x, src, dst)
    h1 = _tc1(x, gd, W_self1, W_neigh1, b1.reshape(1, D))
    (gp2,) = _seg_sum(h1, src, dst)
    wq1 = jnp.pad(W_pred[0:D], ((0, 0), (0, D - 3)))
    wq2 = jnp.pad(W_pred[D:2 * D], ((0, 0), (0, D - 3)))
    wp3 = jnp.pad(W_pred[2 * D:3 * D], ((0, 0), (0, 13)))
    ts, td = _tc2(x, h1, gp2, gd, W_self2, W_neigh2,
                  b2.reshape(1, D), W_ni, W_nj, wq1, wq2)
    a_e, aq, b_e, bq = _edge_gather(ts, td, src, dst)
    out16 = _tc3(z1, a_e, b_e, aq, bq, wp3,
                 jnp.pad(b_pred, (0, 13)).reshape(1, 16))
    return out16[:, :3]


# speedup vs baseline: 1.3996x; 1.3996x over previous
import functools

import jax
import jax.numpy as jnp
from jax import lax
from jax.experimental import pallas as pl
from jax.experimental.pallas import tpu as pltpu
from jax.experimental.pallas import tpu_sc as plsc

N_NODES = 10000
N_EDGES = 320000
D = 128
NC = 2
NS = 16
NW = NC * NS
CH = 80
EPS = N_EDGES // NW
TPS = EPS // CH
HALF = (TPS + 1) // 2
RPT = 624
RTAIL_BASE = NS * RPT
RTAIL = N_NODES - RTAIL_BASE

_MESH = plsc.VectorSubcoreMesh(core_axis_name="c", subcore_axis_name="s")


def _zero_rows(ref, nrows, width):
    def body(i, carry):
        for k in range(width // 16):
            ref[i, pl.ds(k * 16, 16)] = jnp.zeros((16,), jnp.float32)
        return carry
    lax.fori_loop(0, nrows, body, 0)


def _fill_ones_w(ref, nrows, width):
    def body(i, carry):
        for k in range(width // 16):
            ref[i, pl.ds(k * 16, 16)] = jnp.full((16,), 1.0, jnp.float32)
        return carry
    lax.fori_loop(0, nrows, body, 0)


def _copy_zero_region(zbuf, dst, base, total, bufrows):
    nfull = total // bufrows
    rem = total % bufrows
    for r in range(nfull):
        pltpu.sync_copy(zbuf, dst.at[pl.ds(base + r * bufrows, bufrows)])
    if rem:
        pltpu.sync_copy(zbuf.at[pl.ds(0, rem)],
                        dst.at[pl.ds(base + nfull * bufrows, rem)])


def _deg_body(dsti, d_out, ixd, ones, acc):
    c = lax.axis_index("c")
    s = lax.axis_index("s")
    w = s * NC + c
    ebase = w * EPS
    base = s * RPT
    pltpu.sync_copy(dsti.at[pl.ds(ebase, EPS)], ixd)
    _zero_rows(ones, CH, D)
    _copy_zero_region(ones, acc, base, RPT, CH)

    @pl.when(s == NS - 1)
    def _():
        pltpu.sync_copy(ones.at[pl.ds(0, RTAIL)],
                        acc.at[pl.ds(RTAIL_BASE, RTAIL)])

    _fill_ones_w(ones, CH, D)
    plsc.subcore_barrier()

    def step(t, carry):
        pltpu.sync_copy(ones, acc.at[ixd.at[pl.ds(t * CH, CH)]], add=True)
        return carry

    lax.fori_loop(0, TPS, step, 0)
    plsc.subcore_barrier()
    pltpu.sync_copy(acc.at[pl.ds(base, RPT)], d_out.at[c, pl.ds(base, RPT)])

    @pl.when(s == NS - 1)
    def _():
        pltpu.sync_copy(acc.at[pl.ds(RTAIL_BASE, RTAIL)],
                        d_out.at[c, pl.ds(RTAIL_BASE, RTAIL)])


def _seg_body(tbl, srci, dsti, g_out,
              ixs, ixd, rb0, rb1, acc, sg0, sg1):
    c = lax.axis_index("c")
    s = lax.axis_index("s")
    w = s * NC + c
    ebase = w * EPS
    base = s * RPT
    pltpu.sync_copy(srci.at[pl.ds(ebase, EPS)], ixs)
    pltpu.sync_copy(dsti.at[pl.ds(ebase, EPS)], ixd)
    _zero_rows(rb0, CH, D)
    _copy_zero_region(rb0, acc, base, RPT, CH)

    @pl.when(s == NS - 1)
    def _():
        pltpu.sync_copy(rb0.at[pl.ds(0, RTAIL)],
                        acc.at[pl.ds(RTAIL_BASE, RTAIL)])

    plsc.subcore_barrier()

    rbufs = (rb0, rb1)
    sems = (sg0, sg1)

    def issue(chunk, slot):
        pltpu.async_copy(tbl.at[ixs.at[pl.ds(chunk * CH, CH)]],
                         rbufs[slot], sems[slot])

    def drain(slot):
        pltpu.make_async_copy(tbl.at[pl.ds(0, CH)], rbufs[slot],
                              sems[slot]).wait()

    def scat(chunk, slot):
        pltpu.sync_copy(rbufs[slot],
                        acc.at[ixd.at[pl.ds(chunk * CH, CH)]], add=True)

    issue(0, 0)

    def body(i, carry):
        c0 = 2 * i

        @pl.when(c0 + 1 < TPS)
        def _():
            issue(c0 + 1, 1)

        drain(0)
        scat(c0, 0)

        @pl.when(c0 + 1 < TPS)
        def _():
            @pl.when(c0 + 2 < TPS)
            def _():
                issue(c0 + 2, 0)

            drain(1)
            scat(c0 + 1, 1)

        return carry

    lax.fori_loop(0, HALF, body, 0)
    plsc.subcore_barrier()
    pltpu.sync_copy(acc.at[pl.ds(base, RPT)], g_out.at[c, pl.ds(base, RPT)])

    @pl.when(s == NS - 1)
    def _():
        pltpu.sync_copy(acc.at[pl.ds(RTAIL_BASE, RTAIL)],
                        g_out.at[c, pl.ds(RTAIL_BASE, RTAIL)])


_deg_kernel = pl.kernel(
    _deg_body,
    out_type=[jax.ShapeDtypeStruct((NC, N_NODES, D), jnp.float32)],
    mesh=_MESH,
    scratch_types=[
        pltpu.VMEM((EPS,), jnp.int32),
        pltpu.VMEM((CH, D), jnp.float32),
        pltpu.VMEM_SHARED((N_NODES, D), jnp.float32),
    ],
    name="deg_count",
)

_seg_sum = pl.kernel(
    _seg_body,
    out_type=[jax.ShapeDtypeStruct((NC, N_NODES, D), jnp.float32)],
    mesh=_MESH,
    scratch_types=[
        pltpu.VMEM((EPS,), jnp.int32),
        pltpu.VMEM((EPS,), jnp.int32),
        pltpu.VMEM((CH, D), jnp.float32),
        pltpu.VMEM((CH, D), jnp.float32),
        pltpu.VMEM_SHARED((N_NODES, D), jnp.float32),
        pltpu.SemaphoreType.DMA,
        pltpu.SemaphoreType.DMA,
    ],
    name="seg_sum",
)


def _edge_body(ts, td, srci, dsti, a_out, b_out,
               ixs, ixd, ab0, ab1, bb0, bb1, sg0, sg1):
    c = lax.axis_index("c")
    s = lax.axis_index("s")
    w = s * NC + c
    ebase = w * EPS
    pltpu.sync_copy(srci.at[pl.ds(ebase, EPS)], ixs)
    pltpu.sync_copy(dsti.at[pl.ds(ebase, EPS)], ixd)

    abufs = (ab0, ab1)
    bbufs = (bb0, bb1)
    sems = (sg0, sg1)

    def issue(chunk, slot):
        isl = pl.ds(chunk * CH, CH)
        pltpu.async_copy(ts.at[ixs.at[isl]], abufs[slot], sems[slot])
        pltpu.async_copy(td.at[ixd.at[isl]], bbufs[slot], sems[slot])

    def drain(slot):
        pltpu.make_async_copy(ts.at[pl.ds(0, CH)], abufs[slot],
                              sems[slot]).wait()
        pltpu.make_async_copy(td.at[pl.ds(0, CH)], bbufs[slot],
                              sems[slot]).wait()

    def write(chunk, slot):
        osl = pl.ds(ebase + chunk * CH, CH)
        pltpu.sync_copy(abufs[slot], a_out.at[osl])
        pltpu.sync_copy(bbufs[slot], b_out.at[osl])

    issue(0, 0)

    def body(i, carry):
        c0 = 2 * i

        @pl.when(c0 + 1 < TPS)
        def _():
            issue(c0 + 1, 1)

        drain(0)
        write(c0, 0)

        @pl.when(c0 + 1 < TPS)
        def _():
            @pl.when(c0 + 2 < TPS)
            def _():
                issue(c0 + 2, 0)

            drain(1)
            write(c0 + 1, 1)

        return carry

    lax.fori_loop(0, HALF, body, 0)


_edge_gather = pl.kernel(
    _edge_body,
    out_type=[
        jax.ShapeDtypeStruct((N_EDGES, D), jnp.int32),
        jax.ShapeDtypeStruct((N_EDGES, D), jnp.int32),
    ],
    mesh=_MESH,
    scratch_types=[
        pltpu.VMEM((EPS,), jnp.int32),
        pltpu.VMEM((EPS,), jnp.int32),
        pltpu.VMEM((CH, D), jnp.int32),
        pltpu.VMEM((CH, D), jnp.int32),
        pltpu.VMEM((CH, D), jnp.int32),
        pltpu.VMEM((CH, D), jnp.int32),
        pltpu.SemaphoreType.DMA,
        pltpu.SemaphoreType.DMA,
    ],
    name="edge_gather",
)


_RN = 400
_NGRID = N_NODES // _RN
_EB = 512
_EGRID = N_EDGES // _EB


def _tc1_body(x_ref, g_ref, d_ref, ws_ref, wn_ref, b_ref, o_ref):
    g = g_ref[0] + g_ref[1]
    deg = jnp.maximum(d_ref[0, :, :1] + d_ref[1, :, :1], 1.0)
    neigh = g / deg
    h = (jnp.dot(x_ref[...], ws_ref[...], preferred_element_type=jnp.float32)
         + jnp.dot(neigh, wn_ref[...], preferred_element_type=jnp.float32)
         + b_ref[...])
    o_ref[...] = jnp.maximum(h, 0.0)


def _tc1(x, gp, dp, ws, wn, b):
    return pl.pallas_call(
        _tc1_body,
        grid=(_NGRID,),
        in_specs=[
            pl.BlockSpec((_RN, D), lambda i: (i, 0)),
            pl.BlockSpec((NC, _RN, D), lambda i: (0, i, 0)),
            pl.BlockSpec((NC, _RN, D), lambda i: (0, i, 0)),
            pl.BlockSpec((D, D), lambda i: (0, 0)),
            pl.BlockSpec((D, D), lambda i: (0, 0)),
            pl.BlockSpec((1, D), lambda i: (0, 0)),
        ],
        out_specs=pl.BlockSpec((_RN, D), lambda i: (i, 0)),
        out_shape=jax.ShapeDtypeStruct((N_NODES, D), jnp.float32),
    )(x, gp, dp, ws, wn, b)


def _bf16_bits(v):
    b = jax.lax.bitcast_convert_type(v, jnp.int32)
    return ((b + 0x7FFF + ((b >> 16) & 1)) >> 16) & 0xFFFF


def _pack_pair(lo, hi):
    return _bf16_bits(lo) | (_bf16_bits(hi) << 16)


def _tc2_body(x_ref, h1_ref, g_ref, d_ref, ws_ref, wn_ref, b_ref,
              wni_ref, wnj_ref, wq1_ref, wq2_ref,
              ts_ref, td_ref):
    g = g_ref[0] + g_ref[1]
    deg = jnp.maximum(d_ref[0, :, :1] + d_ref[1, :, :1], 1.0)
    neigh = g / deg
    h = (jnp.dot(h1_ref[...], ws_ref[...], preferred_element_type=jnp.float32)
         + jnp.dot(neigh, wn_ref[...], preferred_element_type=jnp.float32)
         + b_ref[...])
    x = x_ref[...]
    fni = jnp.dot(x, wni_ref[...], preferred_element_type=jnp.float32)
    fnj = jnp.dot(x, wnj_ref[...], preferred_element_type=jnp.float32)
    p1 = jnp.dot(h, wq1_ref[...], preferred_element_type=jnp.float32)
    p2 = jnp.dot(h, wq2_ref[...], preferred_element_type=jnp.float32)
    ts_ref[...] = _pack_pair(fni, p1)
    td_ref[...] = _pack_pair(fnj, p2)


def _tc2(x, h1, gp2, dp, ws2, wn2, b2, wni, wnj, wq1, wq2):
    return pl.pallas_call(
        _tc2_body,
        grid=(_NGRID,),
        in_specs=[
            pl.BlockSpec((_RN, D), lambda i: (i, 0)),
            pl.BlockSpec((_RN, D), lambda i: (i, 0)),
            pl.BlockSpec((NC, _RN, D), lambda i: (0, i, 0)),
            pl.BlockSpec((NC, _RN, D), lambda i: (0, i, 0)),
            pl.BlockSpec((D, D), lambda i: (0, 0)),
            pl.BlockSpec((D, D), lambda i: (0, 0)),
            pl.BlockSpec((1, D), lambda i: (0, 0)),
            pl.BlockSpec((D, D), lambda i: (0, 0)),
            pl.BlockSpec((D, D), lambda i: (0, 0)),
            pl.BlockSpec((D, D), lambda i: (0, 0)),
            pl.BlockSpec((D, D), lambda i: (0, 0)),
        ],
        out_specs=[
            pl.BlockSpec((_RN, D), lambda i: (i, 0)),
            pl.BlockSpec((_RN, D), lambda i: (i, 0)),
        ],
        out_shape=[
            jax.ShapeDtypeStruct((N_NODES, D), jnp.int32),
            jax.ShapeDtypeStruct((N_NODES, D), jnp.int32),
        ],
    )(x, h1, gp2, dp, ws2, wn2, b2, wni, wnj, wq1, wq2)


def _unpack_lo(v):
    return jax.lax.bitcast_convert_type(v << 16, jnp.float32)


def _unpack_hi(v):
    return jax.lax.bitcast_convert_type(v & jnp.int32(-65536), jnp.float32)


def _tc3_body(e_ref, a_ref, b_ref, wf_ref, bias_ref, wp3_ref, bp_ref, o_ref):
    ai = a_ref[...]
    bi = b_ref[...]
    s_e = _unpack_lo(ai) + _unpack_lo(bi)
    q_e = _unpack_hi(ai)[:, :16] + _unpack_hi(bi)[:, :16]
    z = (jnp.dot(e_ref[...], wf_ref[...], preferred_element_type=jnp.float32)
         + s_e + bias_ref[...])
    f = jnp.maximum(z, 0.01 * z)
    o_ref[...] = (jnp.dot(f, wp3_ref[...], preferred_element_type=jnp.float32)
                  + q_e + bp_ref[...])


def _tc3(e, a_e, b_e, wf, bias, wp3, bp):
    return pl.pallas_call(
        _tc3_body,
        grid=(_EGRID,),
        in_specs=[
            pl.BlockSpec((_EB, D), lambda i: (i, 0)),
            pl.BlockSpec((_EB, D), lambda i: (i, 0)),
            pl.BlockSpec((_EB, D), lambda i: (i, 0)),
            pl.BlockSpec((D, D), lambda i: (0, 0)),
            pl.BlockSpec((1, D), lambda i: (0, 0)),
            pl.BlockSpec((D, 16), lambda i: (0, 0)),
            pl.BlockSpec((1, 16), lambda i: (0, 0)),
        ],
        out_specs=pl.BlockSpec((_EB, 16), lambda i: (i, 0)),
        out_shape=jax.ShapeDtypeStruct((N_EDGES, 16), jnp.float32),
    )(e, a_e, b_e, wf, bias, wp3, bp)


def kernel(x, e, edge_index, W_self1, W_neigh1, b1, W_self2, W_neigh2, b2,
           W_node_src, W_ni, W_fij, W_nj, egat_bias, attn, W_pred, b_pred):
    src = edge_index[0]
    dst = edge_index[1]
    (dp,) = _deg_kernel(dst)
    (gp,) = _seg_sum(x, src, dst)
    h1 = _tc1(x, gp, dp, W_self1, W_neigh1, b1.reshape(1, D))
    (gp2,) = _seg_sum(h1, src, dst)
    wq1 = jnp.pad(W_pred[0:D], ((0, 0), (0, D - 3)))
    wq2 = jnp.pad(W_pred[D:2 * D], ((0, 0), (0, D - 3)))
    wp3 = jnp.pad(W_pred[2 * D:3 * D], ((0, 0), (0, 13)))
    ts, td = _tc2(x, h1, gp2, dp, W_self2, W_neigh2,
                  b2.reshape(1, D), W_ni, W_nj, wq1, wq2)
    a_e, b_e = _edge_gather(ts, td, src, dst)
    out16 = _tc3(e, a_e, b_e, W_fij, egat_bias.reshape(1, D), wp3,
                 jnp.pad(b_pred, (0, 13)).reshape(1, 16))
    return out16[:, :3]


# speedup vs baseline: 1.4004x; 1.0005x over previous
import jax
import jax.numpy as jnp
from jax import lax
from jax.experimental import pallas as pl
from jax.experimental.pallas import tpu as pltpu
from jax.experimental.pallas import tpu_sc as plsc

N_NODES = 10000
N_EDGES = 320000
D = 128
NC = 2
NS = 16
NW = NC * NS
CH = 80
EPS = N_EDGES // NW
TPS = EPS // CH
HALF = (TPS + 1) // 2
RPT = 624
RTAIL_BASE = NS * RPT
RTAIL = N_NODES - RTAIL_BASE

_MESH = plsc.VectorSubcoreMesh(core_axis_name="c", subcore_axis_name="s")


def _zero_rows(ref, nrows, width):
    def body(i, carry):
        for k in range(width // 16):
            ref[i, pl.ds(k * 16, 16)] = jnp.zeros((16,), jnp.float32)
        return carry
    lax.fori_loop(0, nrows, body, 0)


def _fill_ones_w(ref, nrows, width):
    def body(i, carry):
        for k in range(width // 16):
            ref[i, pl.ds(k * 16, 16)] = jnp.full((16,), 1.0, jnp.float32)
        return carry
    lax.fori_loop(0, nrows, body, 0)


def _copy_zero_region(zbuf, dst, base, total, bufrows):
    nfull = total // bufrows
    rem = total % bufrows
    for r in range(nfull):
        pltpu.sync_copy(zbuf, dst.at[pl.ds(base + r * bufrows, bufrows)])
    if rem:
        pltpu.sync_copy(zbuf.at[pl.ds(0, rem)],
                        dst.at[pl.ds(base + nfull * bufrows, rem)])


def _deg_body(dsti, d_out, ixd, ones, acc):
    c = lax.axis_index("c")
    s = lax.axis_index("s")
    w = s * NC + c
    ebase = w * EPS
    base = s * RPT
    pltpu.sync_copy(dsti.at[pl.ds(ebase, EPS)], ixd)
    _zero_rows(ones, CH, D)
    _copy_zero_region(ones, acc, base, RPT, CH)

    @pl.when(s == NS - 1)
    def _():
        pltpu.sync_copy(ones.at[pl.ds(0, RTAIL)],
                        acc.at[pl.ds(RTAIL_BASE, RTAIL)])

    _fill_ones_w(ones, CH, D)
    plsc.subcore_barrier()

    def step(t, carry):
        pltpu.sync_copy(ones, acc.at[ixd.at[pl.ds(t * CH, CH)]], add=True)
        return carry

    lax.fori_loop(0, TPS, step, 0)
    plsc.subcore_barrier()
    pltpu.sync_copy(acc.at[pl.ds(base, RPT)], d_out.at[c, pl.ds(base, RPT)])

    @pl.when(s == NS - 1)
    def _():
        pltpu.sync_copy(acc.at[pl.ds(RTAIL_BASE, RTAIL)],
                        d_out.at[c, pl.ds(RTAIL_BASE, RTAIL)])


def _seg_body(tbl, srci, dsti, g_out,
              ixs, ixd, rb0, rb1, acc, sg0, sg1):
    c = lax.axis_index("c")
    s = lax.axis_index("s")
    w = s * NC + c
    ebase = w * EPS
    base = s * RPT
    pltpu.sync_copy(srci.at[pl.ds(ebase, EPS)], ixs)
    pltpu.sync_copy(dsti.at[pl.ds(ebase, EPS)], ixd)
    _zero_rows(rb0, CH, D)
    _copy_zero_region(rb0, acc, base, RPT, CH)

    @pl.when(s == NS - 1)
    def _():
        pltpu.sync_copy(rb0.at[pl.ds(0, RTAIL)],
                        acc.at[pl.ds(RTAIL_BASE, RTAIL)])

    plsc.subcore_barrier()

    rbufs = (rb0, rb1)
    sems = (sg0, sg1)

    def issue(chunk, slot):
        pltpu.async_copy(tbl.at[ixs.at[pl.ds(chunk * CH, CH)]],
                         rbufs[slot], sems[slot])

    def drain(slot):
        pltpu.make_async_copy(tbl.at[pl.ds(0, CH)], rbufs[slot],
                              sems[slot]).wait()

    def scat(chunk, slot):
        pltpu.sync_copy(rbufs[slot],
                        acc.at[ixd.at[pl.ds(chunk * CH, CH)]], add=True)

    issue(0, 0)

    def body(i, carry):
        c0 = 2 * i

        @pl.when(c0 + 1 < TPS)
        def _():
            issue(c0 + 1, 1)

        drain(0)
        scat(c0, 0)

        @pl.when(c0 + 1 < TPS)
        def _():
            @pl.when(c0 + 2 < TPS)
            def _():
                issue(c0 + 2, 0)

            drain(1)
            scat(c0 + 1, 1)

        return carry

    lax.fori_loop(0, HALF, body, 0)
    plsc.subcore_barrier()
    pltpu.sync_copy(acc.at[pl.ds(base, RPT)], g_out.at[c, pl.ds(base, RPT)])

    @pl.when(s == NS - 1)
    def _():
        pltpu.sync_copy(acc.at[pl.ds(RTAIL_BASE, RTAIL)],
                        g_out.at[c, pl.ds(RTAIL_BASE, RTAIL)])


_deg_kernel = pl.kernel(
    _deg_body,
    out_type=[jax.ShapeDtypeStruct((NC, N_NODES, D), jnp.float32)],
    mesh=_MESH,
    scratch_types=[
        pltpu.VMEM((EPS,), jnp.int32),
        pltpu.VMEM((CH, D), jnp.float32),
        pltpu.VMEM_SHARED((N_NODES, D), jnp.float32),
    ],
    name="deg_count",
)

_seg_sum = pl.kernel(
    _seg_body,
    out_type=[jax.ShapeDtypeStruct((NC, N_NODES, D), jnp.float32)],
    mesh=_MESH,
    scratch_types=[
        pltpu.VMEM((EPS,), jnp.int32),
        pltpu.VMEM((EPS,), jnp.int32),
        pltpu.VMEM((CH, D), jnp.float32),
        pltpu.VMEM((CH, D), jnp.float32),
        pltpu.VMEM_SHARED((N_NODES, D), jnp.float32),
        pltpu.SemaphoreType.DMA,
        pltpu.SemaphoreType.DMA,
    ],
    name="seg_sum",
)


def _edge_body(ts, td, srci, dsti, a_out, b_out,
               ixs, ixd, ab0, ab1, bb0, bb1, sg0, sg1):
    c = lax.axis_index("c")
    s = lax.axis_index("s")
    w = s * NC + c
    ebase = w * EPS
    pltpu.sync_copy(srci.at[pl.ds(ebase, EPS)], ixs)
    pltpu.sync_copy(dsti.at[pl.ds(ebase, EPS)], ixd)

    abufs = (ab0, ab1)
    bbufs = (bb0, bb1)
    sems = (sg0, sg1)

    def issue(chunk, slot):
        isl = pl.ds(chunk * CH, CH)
        pltpu.async_copy(ts.at[ixs.at[isl]], abufs[slot], sems[slot])
        pltpu.async_copy(td.at[ixd.at[isl]], bbufs[slot], sems[slot])

    def drain(slot):
        pltpu.make_async_copy(ts.at[pl.ds(0, CH)], abufs[slot],
                              sems[slot]).wait()
        pltpu.make_async_copy(td.at[pl.ds(0, CH)], bbufs[slot],
                              sems[slot]).wait()

    def write(chunk, slot):
        osl = pl.ds(ebase + chunk * CH, CH)
        pltpu.sync_copy(abufs[slot], a_out.at[osl])
        pltpu.sync_copy(bbufs[slot], b_out.at[osl])

    issue(0, 0)

    def body(i, carry):
        c0 = 2 * i

        @pl.when(c0 + 1 < TPS)
        def _():
            issue(c0 + 1, 1)

        drain(0)
        write(c0, 0)

        @pl.when(c0 + 1 < TPS)
        def _():
            @pl.when(c0 + 2 < TPS)
            def _():
                issue(c0 + 2, 0)

            drain(1)
            write(c0 + 1, 1)

        return carry

    lax.fori_loop(0, HALF, body, 0)


_edge_gather = pl.kernel(
    _edge_body,
    out_type=[
        jax.ShapeDtypeStruct((N_EDGES, D), jnp.int32),
        jax.ShapeDtypeStruct((N_EDGES, D), jnp.int32),
    ],
    mesh=_MESH,
    scratch_types=[
        pltpu.VMEM((EPS,), jnp.int32),
        pltpu.VMEM((EPS,), jnp.int32),
        pltpu.VMEM((CH, D), jnp.int32),
        pltpu.VMEM((CH, D), jnp.int32),
        pltpu.VMEM((CH, D), jnp.int32),
        pltpu.VMEM((CH, D), jnp.int32),
        pltpu.SemaphoreType.DMA,
        pltpu.SemaphoreType.DMA,
    ],
    name="edge_gather",
)


_RN = 400
_NGRID = N_NODES // _RN
_EB = 512
_EGRID = N_EDGES // _EB


def _tc1_body(x_ref, g_ref, d_ref, ws_ref, wn_ref, b_ref, o_ref):
    g = g_ref[0] + g_ref[1]
    deg = jnp.maximum(d_ref[0, :, :1] + d_ref[1, :, :1], 1.0)
    neigh = g / deg
    h = (jnp.dot(x_ref[...], ws_ref[...], preferred_element_type=jnp.float32)
         + jnp.dot(neigh, wn_ref[...], preferred_element_type=jnp.float32)
         + b_ref[...])
    o_ref[...] = jnp.maximum(h, 0.0)


def _tc1(x, gp, dp, ws, wn, b):
    return pl.pallas_call(
        _tc1_body,
        grid=(_NGRID,),
        in_specs=[
            pl.BlockSpec((_RN, D), lambda i: (i, 0)),
            pl.BlockSpec((NC, _RN, D), lambda i: (0, i, 0)),
            pl.BlockSpec((NC, _RN, D), lambda i: (0, i, 0)),
            pl.BlockSpec((D, D), lambda i: (0, 0)),
            pl.BlockSpec((D, D), lambda i: (0, 0)),
            pl.BlockSpec((1, D), lambda i: (0, 0)),
        ],
        out_specs=pl.BlockSpec((_RN, D), lambda i: (i, 0)),
        out_shape=jax.ShapeDtypeStruct((N_NODES, D), jnp.float32),
    )(x, gp, dp, ws, wn, b)


def _bf16_bits(v):
    b = jax.lax.bitcast_convert_type(v, jnp.int32)
    return ((b + 0x7FFF + ((b >> 16) & 1)) >> 16) & 0xFFFF


def _pack_pair(lo, hi):
    return _bf16_bits(lo) | (_bf16_bits(hi) << 16)


def _tc2_body(x_ref, h1_ref, g_ref, d_ref, ws_ref, wn_ref, b_ref,
              wni_ref, wnj_ref, wq1_ref, wq2_ref,
              ts_ref, td_ref):
    g = g_ref[0] + g_ref[1]
    deg = jnp.maximum(d_ref[0, :, :1] + d_ref[1, :, :1], 1.0)
    neigh = g / deg
    h = (jnp.dot(h1_ref[...], ws_ref[...], preferred_element_type=jnp.float32)
         + jnp.dot(neigh, wn_ref[...], preferred_element_type=jnp.float32)
         + b_ref[...])
    x = x_ref[...]
    fni = jnp.dot(x, wni_ref[...], preferred_element_type=jnp.float32)
    fnj = jnp.dot(x, wnj_ref[...], preferred_element_type=jnp.float32)
    p1 = jnp.dot(h, wq1_ref[...], preferred_element_type=jnp.float32)
    p2 = jnp.dot(h, wq2_ref[...], preferred_element_type=jnp.float32)
    ts_ref[...] = _pack_pair(fni, p1)
    td_ref[...] = _pack_pair(fnj, p2)


def _tc2(x, h1, gp2, dp, ws2, wn2, b2, wni, wnj, wq1, wq2):
    return pl.pallas_call(
        _tc2_body,
        grid=(_NGRID,),
        in_specs=[
            pl.BlockSpec((_RN, D), lambda i: (i, 0)),
            pl.BlockSpec((_RN, D), lambda i: (i, 0)),
            pl.BlockSpec((NC, _RN, D), lambda i: (0, i, 0)),
            pl.BlockSpec((NC, _RN, D), lambda i: (0, i, 0)),
            pl.BlockSpec((D, D), lambda i: (0, 0)),
            pl.BlockSpec((D, D), lambda i: (0, 0)),
            pl.BlockSpec((1, D), lambda i: (0, 0)),
            pl.BlockSpec((D, D), lambda i: (0, 0)),
            pl.BlockSpec((D, D), lambda i: (0, 0)),
            pl.BlockSpec((D, D), lambda i: (0, 0)),
            pl.BlockSpec((D, D), lambda i: (0, 0)),
        ],
        out_specs=[
            pl.BlockSpec((_RN, D), lambda i: (i, 0)),
            pl.BlockSpec((_RN, D), lambda i: (i, 0)),
        ],
        out_shape=[
            jax.ShapeDtypeStruct((N_NODES, D), jnp.int32),
            jax.ShapeDtypeStruct((N_NODES, D), jnp.int32),
        ],
    )(x, h1, gp2, dp, ws2, wn2, b2, wni, wnj, wq1, wq2)


def _unpack_lo(v):
    return jax.lax.bitcast_convert_type(v << 16, jnp.float32)


def _unpack_hi(v):
    return jax.lax.bitcast_convert_type(v & jnp.int32(-65536), jnp.float32)


def _tc3_body(e_ref, a_ref, b_ref, wf_ref, bias_ref, wp3_ref, bp_ref, o_ref):
    ai = a_ref[...]
    bi = b_ref[...]
    s_e = _unpack_lo(ai) + _unpack_lo(bi)
    q_e = _unpack_hi(ai)[:, :16] + _unpack_hi(bi)[:, :16]
    z = (jnp.dot(e_ref[...], wf_ref[...], preferred_element_type=jnp.float32)
         + s_e + bias_ref[...])
    f = jnp.maximum(z, 0.01 * z)
    o_ref[...] = (jnp.dot(f, wp3_ref[...], preferred_element_type=jnp.float32)
                  + q_e + bp_ref[...])


def _tc3(e, a_e, b_e, wf, bias, wp3, bp):
    return pl.pallas_call(
        _tc3_body,
        grid=(_EGRID,),
        in_specs=[
            pl.BlockSpec((_EB, D), lambda i: (i, 0)),
            pl.BlockSpec((_EB, D), lambda i: (i, 0)),
            pl.BlockSpec((_EB, D), lambda i: (i, 0)),
            pl.BlockSpec((D, D), lambda i: (0, 0)),
            pl.BlockSpec((1, D), lambda i: (0, 0)),
            pl.BlockSpec((D, 16), lambda i: (0, 0)),
            pl.BlockSpec((1, 16), lambda i: (0, 0)),
        ],
        out_specs=pl.BlockSpec((_EB, 16), lambda i: (i, 0)),
        out_shape=jax.ShapeDtypeStruct((N_EDGES, 16), jnp.float32),
    )(e, a_e, b_e, wf, bias, wp3, bp)


def kernel(x, e, edge_index, W_self1, W_neigh1, b1, W_self2, W_neigh2, b2,
           W_node_src, W_ni, W_fij, W_nj, egat_bias, attn, W_pred, b_pred):
    src = edge_index[0]
    dst = edge_index[1]
    (dp,) = _deg_kernel(dst)
    (gp,) = _seg_sum(x, src, dst)
    h1 = _tc1(x, gp, dp, W_self1, W_neigh1, b1.reshape(1, D))
    (gp2,) = _seg_sum(h1, src, dst)
    wq1 = jnp.pad(W_pred[0:D], ((0, 0), (0, D - 3)))
    wq2 = jnp.pad(W_pred[D:2 * D], ((0, 0), (0, D - 3)))
    wp3 = jnp.pad(W_pred[2 * D:3 * D], ((0, 0), (0, 13)))
    ts, td = _tc2(x, h1, gp2, dp, W_self2, W_neigh2,
                  b2.reshape(1, D), W_ni, W_nj, wq1, wq2)
    a_e, b_e = _edge_gather(ts, td, src, dst)
    out16 = _tc3(e, a_e, b_e, W_fij, egat_bias.reshape(1, D), wp3,
                 jnp.pad(b_pred, (0, 13)).reshape(1, 16))
    return out16[:, :3]
